# Initial kernel scaffold; baseline (speedup 1.0000x reference)
import jax
import jax.numpy as jnp
from jax.experimental import pallas as pl


def kernel(ids, emb_table, w1, b1, w2, b2):
    raise NotImplementedError("write your pallas kernel here")


# speedup vs baseline: 3.8049x; 3.8049x over previous
import jax
import jax.numpy as jnp
from jax import lax
from jax.experimental import pallas as pl
from jax.experimental.pallas import tpu as pltpu

VOCAB = 32
EMB = 16
CTX = 2
HID = 128
NEG_BIG = -1e30


def _ngram_kernel(ids_ref, w1f_ref, b1_ref, w2t_ref, b2_ref, out_ref):
    tb = ids_ref.shape[0]
    ids = ids_ref[...]

    col = lax.broadcasted_iota(jnp.int32, (tb, CTX * VOCAB), 1)
    hit = (col == ids[:, 0:1]) | (col == ids[:, 1:2] + VOCAB)
    onehot = hit.astype(jnp.bfloat16)

    h = jnp.dot(onehot, w1f_ref[...], preferred_element_type=jnp.float32)
    h = jnp.maximum(h + b1_ref[...], 0.0)

    logits = jnp.dot(h.astype(jnp.bfloat16), w2t_ref[...],
                     preferred_element_type=jnp.float32) + b2_ref[...]

    m = jnp.max(logits, axis=-1, keepdims=True)
    lse = jnp.log(jnp.sum(jnp.exp(logits - m), axis=-1, keepdims=True)) + m
    out_ref[...] = (logits - lse)[:, :VOCAB]


def kernel(ids, emb_table, w1, b1, w2, b2, *, tb=1024):
    B = ids.shape[0]
    n_tiles = pl.cdiv(B, tb)
    Bp = n_tiles * tb
    if Bp != B:
        ids = jnp.pad(ids, ((0, Bp - B), (0, 0)))

    w1t = jnp.transpose(w1).reshape(CTX, EMB, HID)
    w1fold = jnp.einsum('ve,ceh->cvh', emb_table.astype(jnp.float32),
                        w1t).reshape(CTX * VOCAB, HID)
    w1f_bf = w1fold.astype(jnp.bfloat16)
    b1_2d = b1.reshape(1, HID)
    w2t = jnp.zeros((HID, HID), jnp.float32).at[:, :VOCAB].set(jnp.transpose(w2))
    w2t_bf = w2t.astype(jnp.bfloat16)
    b2_2d = jnp.full((1, HID), NEG_BIG, jnp.float32).at[0, :VOCAB].set(b2)

    out = pl.pallas_call(
        _ngram_kernel,
        out_shape=jax.ShapeDtypeStruct((Bp, VOCAB), jnp.float32),
        grid=(n_tiles,),
        in_specs=[
            pl.BlockSpec((tb, CTX), lambda i: (i, 0)),
            pl.BlockSpec((CTX * VOCAB, HID), lambda i: (0, 0)),
            pl.BlockSpec((1, HID), lambda i: (0, 0)),
            pl.BlockSpec((HID, HID), lambda i: (0, 0)),
            pl.BlockSpec((1, HID), lambda i: (0, 0)),
        ],
        out_specs=pl.BlockSpec((tb, VOCAB), lambda i: (i, 0)),
        compiler_params=pltpu.CompilerParams(
            dimension_semantics=("parallel",)),
    )(ids, w1f_bf, b1_2d, w2t_bf, b2_2d)

    return out[:B]


# speedup vs baseline: 5.2842x; 1.3888x over previous
import jax
import jax.numpy as jnp
from jax import lax
from jax.experimental import pallas as pl
from jax.experimental.pallas import tpu as pltpu

VOCAB = 32
EMB = 16
CTX = 2
HID = 128
NEG_BIG = -1e30


def _ngram_kernel(ids_ref, w1f_ref, b1_ref, w2t_ref, b2_ref, out_ref):
    tb = ids_ref.shape[0]
    ids = ids_ref[...]

    col = lax.broadcasted_iota(jnp.int32, (tb, CTX * VOCAB), 1)
    hit = (col == ids[:, 0:1]) | (col == ids[:, 1:2] + VOCAB)
    onehot = hit.astype(jnp.bfloat16)

    h = jnp.dot(onehot, w1f_ref[...], preferred_element_type=jnp.float32)
    h = jnp.maximum(h + b1_ref[...], 0.0)

    logits = jnp.dot(h.astype(jnp.bfloat16), w2t_ref[...],
                     preferred_element_type=jnp.float32) + b2_ref[...]

    m = jnp.max(logits, axis=-1, keepdims=True)
    lse = jnp.log(jnp.sum(jnp.exp(logits - m), axis=-1, keepdims=True)) + m
    out_ref[...] = (logits - lse)[:, :VOCAB]


def kernel(ids, emb_table, w1, b1, w2, b2, *, tb=4096):
    B = ids.shape[0]
    n_tiles = pl.cdiv(B, tb)
    Bp = n_tiles * tb
    if Bp != B:
        ids = jnp.pad(ids, ((0, Bp - B), (0, 0)))

    w1t = jnp.transpose(w1).reshape(CTX, EMB, HID)
    w1fold = jnp.einsum('ve,ceh->cvh', emb_table.astype(jnp.float32),
                        w1t).reshape(CTX * VOCAB, HID)
    w1f_bf = w1fold.astype(jnp.bfloat16)
    b1_2d = b1.reshape(1, HID)
    w2t = jnp.zeros((HID, HID), jnp.float32).at[:, :VOCAB].set(jnp.transpose(w2))
    w2t_bf = w2t.astype(jnp.bfloat16)
    b2_2d = jnp.full((1, HID), NEG_BIG, jnp.float32).at[0, :VOCAB].set(b2)

    out = pl.pallas_call(
        _ngram_kernel,
        out_shape=jax.ShapeDtypeStruct((Bp, VOCAB), jnp.float32),
        grid=(n_tiles,),
        in_specs=[
            pl.BlockSpec((tb, CTX), lambda i: (i, 0)),
            pl.BlockSpec((CTX * VOCAB, HID), lambda i: (0, 0)),
            pl.BlockSpec((1, HID), lambda i: (0, 0)),
            pl.BlockSpec((HID, HID), lambda i: (0, 0)),
            pl.BlockSpec((1, HID), lambda i: (0, 0)),
        ],
        out_specs=pl.BlockSpec((tb, VOCAB), lambda i: (i, 0)),
        compiler_params=pltpu.CompilerParams(
            dimension_semantics=("parallel",)),
    )(ids, w1f_bf, b1_2d, w2t_bf, b2_2d)

    return out[:B]


# speedup vs baseline: 9.9474x; 1.8825x over previous
import jax
import jax.numpy as jnp
from jax import lax
from jax.experimental import pallas as pl
from jax.experimental.pallas import tpu as pltpu

VOCAB = 32
EMB = 16
CTX = 2
HID = 128
K1 = 80
K2 = 144


def _ngram_kernel(flat_ref, w1_ref, w2_ref, out_ref):
    tb = out_ref.shape[0]
    flat = flat_ref[0]
    id0 = flat >> 5
    id1 = flat & (VOCAB - 1)

    row = lax.broadcasted_iota(jnp.int32, (K1, tb), 0)
    hit = (row == id0) | (row == id1 + VOCAB) | (row == 2 * VOCAB)
    onehot = hit.astype(jnp.bfloat16)

    h = jnp.dot(w1_ref[...], onehot,
                preferred_element_type=jnp.float32)
    h = jnp.maximum(h, 0.0).astype(jnp.bfloat16)

    ones = jnp.ones((K2 - HID, tb), jnp.bfloat16)
    h2 = jnp.concatenate([h, ones], axis=0)
    logits = jnp.dot(w2_ref[...], h2,
                     preferred_element_type=jnp.float32)

    m = jnp.max(logits, axis=0, keepdims=True)
    lse = jnp.log(jnp.sum(jnp.exp(logits - m), axis=0, keepdims=True)) + m
    out_ref[...] = jnp.transpose(logits - lse, (1, 0))


def kernel(ids, emb_table, w1, b1, w2, b2, *, tb=4096):
    B = ids.shape[0]
    n_tiles = pl.cdiv(B, tb)
    Bp = n_tiles * tb
    if Bp != B:
        ids = jnp.pad(ids, ((0, Bp - B), (0, 0)))

    flat = (ids[:, 0] * VOCAB + ids[:, 1]).reshape(n_tiles, 1, tb)

    w1t = jnp.transpose(w1).reshape(CTX, EMB, HID)
    w1fold = jnp.einsum('ve,ceh->cvh', emb_table.astype(jnp.float32),
                        w1t).reshape(CTX * VOCAB, HID)
    w1aug = jnp.zeros((HID, K1), jnp.float32)
    w1aug = w1aug.at[:, :CTX * VOCAB].set(jnp.transpose(w1fold))
    w1aug = w1aug.at[:, CTX * VOCAB].set(b1)
    w2aug = jnp.zeros((VOCAB, K2), jnp.float32)
    w2aug = w2aug.at[:, :HID].set(w2)
    w2aug = w2aug.at[:, HID].set(b2)

    out = pl.pallas_call(
        _ngram_kernel,
        out_shape=jax.ShapeDtypeStruct((Bp, VOCAB), jnp.float32),
        grid=(n_tiles,),
        in_specs=[
            pl.BlockSpec((1, 1, tb), lambda i: (i, 0, 0)),
            pl.BlockSpec((HID, K1), lambda i: (0, 0)),
            pl.BlockSpec((VOCAB, K2), lambda i: (0, 0)),
        ],
        out_specs=pl.BlockSpec((tb, VOCAB), lambda i: (i, 0)),
        compiler_params=pltpu.CompilerParams(
            dimension_semantics=("parallel",)),
    )(flat, w1aug.astype(jnp.bfloat16), w2aug.astype(jnp.bfloat16))

    return out[:B]


# speedup vs baseline: 11.0051x; 1.1063x over previous
import jax
import jax.numpy as jnp
from jax import lax
from jax.experimental import pallas as pl
from jax.experimental.pallas import tpu as pltpu

VOCAB = 32
EMB = 16
CTX = 2
HID = 128
K1 = 80
K2 = 144


def _ngram_kernel(flat_ref, w1_ref, w2_ref, out_ref):
    tb = out_ref.shape[0]
    flat = flat_ref[0]
    id0 = flat >> 5
    id1 = flat & (VOCAB - 1)

    row = lax.broadcasted_iota(jnp.int32, (K1, tb), 0)
    hit = (row == id0) | (row == id1 + VOCAB) | (row == 2 * VOCAB)
    onehot = hit.astype(jnp.bfloat16)

    h = jnp.dot(w1_ref[...], onehot,
                preferred_element_type=jnp.float32)
    h = jnp.maximum(h, 0.0).astype(jnp.bfloat16)

    ones = jnp.ones((K2 - HID, tb), jnp.bfloat16)
    h2 = jnp.concatenate([h, ones], axis=0)
    logits = jnp.dot(w2_ref[...], h2,
                     preferred_element_type=jnp.float32)

    m = jnp.max(logits, axis=0, keepdims=True)
    lse = jnp.log(jnp.sum(jnp.exp(logits - m), axis=0, keepdims=True)) + m
    out_ref[...] = jnp.transpose(logits - lse, (1, 0))


def kernel(ids, emb_table, w1, b1, w2, b2, *, tb=8192):
    B = ids.shape[0]
    n_tiles = pl.cdiv(B, tb)
    Bp = n_tiles * tb
    if Bp != B:
        ids = jnp.pad(ids, ((0, Bp - B), (0, 0)))

    flat = (ids[:, 0] * VOCAB + ids[:, 1]).reshape(n_tiles, 1, tb)

    w1t = jnp.transpose(w1).reshape(CTX, EMB, HID)
    w1fold = jnp.einsum('ve,ceh->cvh', emb_table.astype(jnp.float32),
                        w1t).reshape(CTX * VOCAB, HID)
    w1aug = jnp.zeros((HID, K1), jnp.float32)
    w1aug = w1aug.at[:, :CTX * VOCAB].set(jnp.transpose(w1fold))
    w1aug = w1aug.at[:, CTX * VOCAB].set(b1)
    w2aug = jnp.zeros((VOCAB, K2), jnp.float32)
    w2aug = w2aug.at[:, :HID].set(w2)
    w2aug = w2aug.at[:, HID].set(b2)

    out = pl.pallas_call(
        _ngram_kernel,
        out_shape=jax.ShapeDtypeStruct((Bp, VOCAB), jnp.float32),
        grid=(n_tiles,),
        in_specs=[
            pl.BlockSpec((1, 1, tb), lambda i: (i, 0, 0)),
            pl.BlockSpec((HID, K1), lambda i: (0, 0)),
            pl.BlockSpec((VOCAB, K2), lambda i: (0, 0)),
        ],
        out_specs=pl.BlockSpec((tb, VOCAB), lambda i: (i, 0)),
        compiler_params=pltpu.CompilerParams(
            dimension_semantics=("parallel",)),
    )(flat, w1aug.astype(jnp.bfloat16), w2aug.astype(jnp.bfloat16))

    return out[:B]


# speedup vs baseline: 11.4117x; 1.0369x over previous
import jax
import jax.numpy as jnp
from jax import lax
from jax.experimental import pallas as pl
from jax.experimental.pallas import tpu as pltpu

VOCAB = 32
EMB = 16
CTX = 2
HID = 128
K1 = 80
K2 = 144


def _ngram_kernel(flat_ref, w1_ref, w2_ref, out_ref):
    tb = out_ref.shape[0]
    flat = flat_ref[0]
    id0 = flat >> 5
    id1 = flat & (VOCAB - 1)

    row = lax.broadcasted_iota(jnp.int32, (K1, tb), 0)
    hit = (row == id0) | (row == id1 + VOCAB) | (row == 2 * VOCAB)
    onehot = hit.astype(jnp.bfloat16)

    h = jnp.dot(w1_ref[...], onehot,
                preferred_element_type=jnp.float32)
    h = jnp.maximum(h, 0.0).astype(jnp.bfloat16)

    ones = jnp.ones((K2 - HID, tb), jnp.bfloat16)
    h2 = jnp.concatenate([h, ones], axis=0)
    logits = jnp.dot(w2_ref[...], h2,
                     preferred_element_type=jnp.float32)

    m = jnp.max(logits, axis=0, keepdims=True)
    lse = jnp.log(jnp.sum(jnp.exp(logits - m), axis=0, keepdims=True)) + m
    out_ref[...] = jnp.transpose(logits - lse, (1, 0))


def kernel(ids, emb_table, w1, b1, w2, b2, *, tb=16384):
    B = ids.shape[0]
    n_tiles = pl.cdiv(B, tb)
    Bp = n_tiles * tb
    if Bp != B:
        ids = jnp.pad(ids, ((0, Bp - B), (0, 0)))

    flat = (ids[:, 0] * VOCAB + ids[:, 1]).reshape(n_tiles, 1, tb)

    w1t = jnp.transpose(w1).reshape(CTX, EMB, HID)
    w1fold = jnp.einsum('ve,ceh->cvh', emb_table.astype(jnp.float32),
                        w1t).reshape(CTX * VOCAB, HID)
    w1aug = jnp.zeros((HID, K1), jnp.float32)
    w1aug = w1aug.at[:, :CTX * VOCAB].set(jnp.transpose(w1fold))
    w1aug = w1aug.at[:, CTX * VOCAB].set(b1)
    w2aug = jnp.zeros((VOCAB, K2), jnp.float32)
    w2aug = w2aug.at[:, :HID].set(w2)
    w2aug = w2aug.at[:, HID].set(b2)

    out = pl.pallas_call(
        _ngram_kernel,
        out_shape=jax.ShapeDtypeStruct((Bp, VOCAB), jnp.float32),
        grid=(n_tiles,),
        in_specs=[
            pl.BlockSpec((1, 1, tb), lambda i: (i, 0, 0)),
            pl.BlockSpec((HID, K1), lambda i: (0, 0)),
            pl.BlockSpec((VOCAB, K2), lambda i: (0, 0)),
        ],
        out_specs=pl.BlockSpec((tb, VOCAB), lambda i: (i, 0)),
        compiler_params=pltpu.CompilerParams(
            dimension_semantics=("parallel",)),
    )(flat, w1aug.astype(jnp.bfloat16), w2aug.astype(jnp.bfloat16))

    return out[:B]
